# baseline (speedup 1.0000x reference)
import jax
import jax.numpy as jnp
import numpy as np
from jax.experimental import pallas as pl
from jax.experimental.pallas import tpu as pltpu

_LANES = 512
_N = 4096
_I32_MIN = np.int32(-2147483648)
_I32_MAX = np.int32(2147483647)


def _select_body(mt_ref, th_ref, idx_ref):
    mt = mt_ref[...]

    valid = mt != jnp.inf

    s = jax.lax.bitcast_convert_type(mt, jnp.int32)
    key = jnp.where(s >= 0, s, s ^ _I32_MAX)
    key = jnp.where(key == jnp.int32(-1), jnp.int32(0), key)
    key = jnp.where(mt != mt, _I32_MAX, key)

    n_valid = jnp.sum(valid.astype(jnp.int32), axis=0, keepdims=True)
    k = (n_valid - 1) // 2

    lo = jnp.full(k.shape, _I32_MIN, jnp.int32)
    for bit in range(31, -1, -1):
        if bit == 31:
            mid = jnp.zeros(k.shape, jnp.int32)
        else:
            mid = lo | jnp.int32(1 << bit)
        cnt = jnp.sum((key < mid).astype(jnp.int32), axis=0, keepdims=True)
        lo = jnp.where(cnt <= k, mid, lo)

    cnt_less = jnp.sum((key < lo).astype(jnp.int32), axis=0, keepdims=True)
    j = k - cnt_less
    iota = jax.lax.broadcasted_iota(jnp.int32, key.shape, 0)
    eqi = jnp.where(key == lo, iota, jnp.int32(_N))

    def _first_eq(_):
        return jnp.min(eqi, axis=0, keepdims=True)

    def _rank_j(_):
        loi = jnp.zeros(k.shape, jnp.int32)
        for bit in range(11, -1, -1):
            mid = loi | jnp.int32(1 << bit)
            cnt = jnp.sum((eqi < mid).astype(jnp.int32), axis=0,
                          keepdims=True)
            loi = jnp.where(cnt <= j, mid, loi)
        return loi

    loi = jax.lax.cond(jnp.any(j > 0), _rank_j, _first_eq, 0)

    srec = jnp.where(lo >= 0, lo, lo ^ _I32_MAX)
    th_ref[...] = jax.lax.bitcast_convert_type(srec, jnp.float32)
    idx_ref[...] = loi


def _median_select(mt_t, interpret=False):
    b = mt_t.shape[1]
    spec = pl.BlockSpec((_N, _LANES), lambda i: (0, i))
    out_spec = pl.BlockSpec((1, _LANES), lambda i: (0, i))
    return pl.pallas_call(
        _select_body,
        grid=(b // _LANES,),
        in_specs=[spec],
        out_specs=[out_spec, out_spec],
        out_shape=[
            jax.ShapeDtypeStruct((1, b), jnp.float32),
            jax.ShapeDtypeStruct((1, b), jnp.int32),
        ],
        compiler_params=pltpu.CompilerParams(
            dimension_semantics=("parallel",),
        ),
        interpret=interpret,
    )(mt_t)


def _theta_plane(y2, y1, x2, x1):
    r1 = jnp.sqrt(x1 ** 2 + y1 ** 2)
    r2 = jnp.sqrt(x2 ** 2 + y2 ** 2)
    a1 = jnp.arctan2(y1, x1)
    a2 = jnp.arctan2(y2, x2)
    thetas = 2.0 * jnp.arctan(
        (-jnp.sin(a2) + (r1 / r2) * jnp.sin(a1))
        / ((r1 / r2) * jnp.cos(a1) + jnp.cos(a2))
    )
    return thetas


def _curvature_at(y2, y1, x2, x1):
    r1 = jnp.sqrt(x1 ** 2 + y1 ** 2)
    r2 = jnp.sqrt(x2 ** 2 + y2 ** 2)
    a1 = jnp.arctan2(y1, x1)
    a2 = jnp.arctan2(y2, x2)
    thetas = _theta_plane(y2, y1, x2, x1)
    stationary = (r1 == r2) & (a1 == a2)
    radii = r2 * jnp.sin(a1 - a2 - thetas) / (
        2.0 * jnp.sin(thetas / 2.0) * jnp.sin(-a1 + thetas / 2.0)
    )
    radii = jnp.where(stationary, jnp.inf, radii)
    return 1.0 / radii


def kernel(x, interpret=False):
    b = x.shape[0]
    xt = jnp.transpose(x, (2, 1, 0))
    y2 = xt[0]
    y1 = xt[1]
    x2 = xt[2]
    x1 = xt[3]
    validity = (y2 != 0.0) | (y1 != 0.0) | (x2 != 0.0) | (x1 != 0.0)

    thetas = _theta_plane(y2, y1, x2, x1)
    mt = jnp.where(validity, thetas, jnp.inf)

    th_est, med_idx = _median_select(mt, interpret=interpret)

    g = jnp.take_along_axis(x, med_idx.reshape(b, 1, 1), axis=1)
    cv_est = _curvature_at(g[:, 0, 0], g[:, 0, 1], g[:, 0, 2], g[:, 0, 3])

    return jnp.stack([th_est.reshape(b), cv_est.reshape(b)], axis=1)


# speedup vs baseline: 1.0511x; 1.0511x over previous
import jax
import jax.numpy as jnp
import numpy as np
from jax.experimental import pallas as pl
from jax.experimental.pallas import tpu as pltpu

_LANES = 256
_N = 4096
_I32_MIN = np.int32(-2147483648)
_I32_MAX = np.int32(2147483647)


def _colsum(v):
    m = v.shape[0]
    while m > 8:
        h = m // 2
        v = v[:h] + v[h:]
        m = h
    return jnp.sum(v, axis=0, keepdims=True)


def _colmin(v):
    m = v.shape[0]
    while m > 8:
        h = m // 2
        v = jnp.minimum(v[:h], v[h:])
        m = h
    return jnp.min(v, axis=0, keepdims=True)


def _select_body(mt_ref, th_ref, idx_ref):
    mt = mt_ref[...]

    valid = mt != jnp.inf

    s = jax.lax.bitcast_convert_type(mt, jnp.int32)
    key = jnp.where(s >= 0, s, s ^ _I32_MAX)
    key = jnp.where(key == jnp.int32(-1), jnp.int32(0), key)
    key = jnp.where(mt != mt, _I32_MAX, key)

    n_valid = _colsum(valid.astype(jnp.int32))
    k = (n_valid - 1) // 2

    lo = jnp.full(k.shape, _I32_MIN, jnp.int32)
    for bit in range(31, -1, -1):
        if bit == 31:
            mid = jnp.zeros(k.shape, jnp.int32)
        else:
            mid = lo | jnp.int32(1 << bit)
        cnt = _colsum((key < mid).astype(jnp.int32))
        lo = jnp.where(cnt <= k, mid, lo)

    cnt_less = _colsum((key < lo).astype(jnp.int32))
    j = k - cnt_less
    iota = jax.lax.broadcasted_iota(jnp.int32, key.shape, 0)
    eqi = jnp.where(key == lo, iota, jnp.int32(_N))

    def _first_eq(_):
        return _colmin(eqi)

    def _rank_j(_):
        loi = jnp.zeros(k.shape, jnp.int32)
        for bit in range(11, -1, -1):
            mid = loi | jnp.int32(1 << bit)
            cnt = _colsum((eqi < mid).astype(jnp.int32))
            loi = jnp.where(cnt <= j, mid, loi)
        return loi

    loi = jax.lax.cond(jnp.any(j > 0), _rank_j, _first_eq, 0)

    srec = jnp.where(lo >= 0, lo, lo ^ _I32_MAX)
    th_ref[...] = jax.lax.bitcast_convert_type(srec, jnp.float32)
    idx_ref[...] = loi


def _median_select(mt_t, interpret=False):
    b = mt_t.shape[1]
    spec = pl.BlockSpec((_N, _LANES), lambda i: (0, i))
    out_spec = pl.BlockSpec((1, _LANES), lambda i: (0, i))
    return pl.pallas_call(
        _select_body,
        grid=(b // _LANES,),
        in_specs=[spec],
        out_specs=[out_spec, out_spec],
        out_shape=[
            jax.ShapeDtypeStruct((1, b), jnp.float32),
            jax.ShapeDtypeStruct((1, b), jnp.int32),
        ],
        compiler_params=pltpu.CompilerParams(
            dimension_semantics=("parallel",),
        ),
        interpret=interpret,
    )(mt_t)


def _theta_plane(y2, y1, x2, x1):
    r1 = jnp.sqrt(x1 ** 2 + y1 ** 2)
    r2 = jnp.sqrt(x2 ** 2 + y2 ** 2)
    a1 = jnp.arctan2(y1, x1)
    a2 = jnp.arctan2(y2, x2)
    thetas = 2.0 * jnp.arctan(
        (-jnp.sin(a2) + (r1 / r2) * jnp.sin(a1))
        / ((r1 / r2) * jnp.cos(a1) + jnp.cos(a2))
    )
    return thetas


def _curvature_at(y2, y1, x2, x1):
    r1 = jnp.sqrt(x1 ** 2 + y1 ** 2)
    r2 = jnp.sqrt(x2 ** 2 + y2 ** 2)
    a1 = jnp.arctan2(y1, x1)
    a2 = jnp.arctan2(y2, x2)
    thetas = _theta_plane(y2, y1, x2, x1)
    stationary = (r1 == r2) & (a1 == a2)
    radii = r2 * jnp.sin(a1 - a2 - thetas) / (
        2.0 * jnp.sin(thetas / 2.0) * jnp.sin(-a1 + thetas / 2.0)
    )
    radii = jnp.where(stationary, jnp.inf, radii)
    return 1.0 / radii


def kernel(x, interpret=False):
    b = x.shape[0]
    xt = jnp.transpose(x, (2, 1, 0))
    y2 = xt[0]
    y1 = xt[1]
    x2 = xt[2]
    x1 = xt[3]
    validity = (y2 != 0.0) | (y1 != 0.0) | (x2 != 0.0) | (x1 != 0.0)

    thetas = _theta_plane(y2, y1, x2, x1)
    mt = jnp.where(validity, thetas, jnp.inf)

    th_est, med_idx = _median_select(mt, interpret=interpret)

    g = jnp.take_along_axis(x, med_idx.reshape(b, 1, 1), axis=1)
    cv_est = _curvature_at(g[:, 0, 0], g[:, 0, 1], g[:, 0, 2], g[:, 0, 3])

    return jnp.stack([th_est.reshape(b), cv_est.reshape(b)], axis=1)
